# baseline (speedup 1.0000x reference)
import functools

import jax
import jax.numpy as jnp
from jax import lax
from jax.experimental import pallas as pl
from jax.experimental.pallas import tpu as pltpu
from jax.experimental.pallas import tpu_sc as plsc

BATCH = 16384
EMBED_K = 32
LANES = 16
NUM_WORKERS = 32
ROWS_PER_W = BATCH // NUM_WORKERS
CHUNK = 128
NCHUNK = ROWS_PER_W // CHUNK
GROUPS = ROWS_PER_W // LANES


def _sc_body(uidx_hbm, iidx_hbm, w_hbm, h_hbm, out_hbm,
             idx_u, idx_i, u_rows, v_rows, out_v, sem):
    nc = 2
    wid = lax.axis_index("s") * nc + lax.axis_index("c")

    pltpu.sync_copy(uidx_hbm.at[pl.ds(wid * NCHUNK, NCHUNK)], idx_u)
    pltpu.sync_copy(iidx_hbm.at[pl.ds(wid * NCHUNK, NCHUNK)], idx_i)

    copies = []
    for j in range(NCHUNK):
        copies.append(pltpu.async_copy(
            w_hbm.at[idx_u.at[j]], u_rows.at[pl.ds(j * CHUNK, CHUNK)], sem))
        copies.append(pltpu.async_copy(
            h_hbm.at[idx_i.at[j]], v_rows.at[pl.ds(j * CHUNK, CHUNK)], sem))
    for c in copies:
        c.wait()

    lane = lax.iota(jnp.int32, LANES)

    def group_body(g, _):
        acc = jnp.zeros((LANES,), jnp.float32)
        for rr in range(LANES):
            r = g * LANES + rr
            u0 = u_rows[r, pl.ds(0, LANES)]
            u1 = u_rows[r, pl.ds(LANES, LANES)]
            v0 = v_rows[r, pl.ds(0, LANES)]
            v1 = v_rows[r, pl.ds(LANES, LANES)]
            s = jnp.sum(u0 * v0 + u1 * v1)
            acc = jnp.where(lane == rr, s, acc)
        out_v[pl.ds(g * LANES, LANES)] = acc
        return _

    lax.fori_loop(0, GROUPS, group_body, 0)

    pltpu.sync_copy(out_v, out_hbm.at[pl.ds(wid * ROWS_PER_W, ROWS_PER_W)])


@jax.jit
def kernel(x, W, H):
    uidx = x[:, 0].reshape(NUM_WORKERS * NCHUNK, CHUNK).astype(jnp.int32)
    iidx = x[:, 1].reshape(NUM_WORKERS * NCHUNK, CHUNK).astype(jnp.int32)

    mesh = plsc.VectorSubcoreMesh(core_axis_name="c", subcore_axis_name="s")
    run = functools.partial(
        pl.kernel,
        out_type=jax.ShapeDtypeStruct((BATCH,), jnp.float32),
        mesh=mesh,
        compiler_params=pltpu.CompilerParams(
            needs_layout_passes=False, use_tc_tiling_on_sc=False),
        scratch_types=[
            pltpu.VMEM((NCHUNK, CHUNK), jnp.int32),
            pltpu.VMEM((NCHUNK, CHUNK), jnp.int32),
            pltpu.VMEM((ROWS_PER_W, EMBED_K), jnp.float32),
            pltpu.VMEM((ROWS_PER_W, EMBED_K), jnp.float32),
            pltpu.VMEM((ROWS_PER_W,), jnp.float32),
            pltpu.SemaphoreType.DMA,
        ],
    )(_sc_body)
    return run(uidx, iidx, W, H)


# speedup vs baseline: 1.5012x; 1.5012x over previous
import functools

import jax
import jax.numpy as jnp
from jax import lax
from jax.experimental import pallas as pl
from jax.experimental.pallas import tpu as pltpu
from jax.experimental.pallas import tpu_sc as plsc

BATCH = 16384
EMBED_K = 32
LANES = 16
NUM_WORKERS = 32
ROWS_PER_W = BATCH // NUM_WORKERS
GROUPS = ROWS_PER_W // LANES
WINDOW_STEPS = 2


def _sc_body(uidx_hbm, iidx_hbm, w_hbm, h_hbm, out_hbm,
             idx_u, idx_i, rows, out_v, sem):
    nc = 2
    wid = lax.axis_index("s") * nc + lax.axis_index("c")
    base = wid * ROWS_PER_W

    pltpu.sync_copy(uidx_hbm.at[pl.ds(base, ROWS_PER_W)], idx_u)
    pltpu.sync_copy(iidx_hbm.at[pl.ds(base, ROWS_PER_W)], idx_i)

    def fire_step(t):
        vu = idx_u[pl.ds(t * LANES, LANES)]
        vi = idx_i[pl.ds(t * LANES, LANES)]
        for k in range(LANES):
            j = t * LANES + k
            pltpu.async_copy(
                w_hbm.at[vu[k]], rows.at[j, pl.ds(0, EMBED_K)], sem)
            pltpu.async_copy(
                h_hbm.at[vi[k]], rows.at[j, pl.ds(EMBED_K, EMBED_K)], sem)

    def drain_step():
        for _ in range(2 * LANES):
            pltpu.make_async_copy(
                w_hbm.at[0], rows.at[0, pl.ds(0, EMBED_K)], sem).wait()

    nsteps = ROWS_PER_W // LANES
    def fire_all(t, c):
        fire_step(t)
        return c
    lax.fori_loop(0, nsteps, fire_all, 0)

    def final_drain(t, c):
        drain_step()
        return c
    lax.fori_loop(0, nsteps, final_drain, 0)

    lane = lax.iota(jnp.int32, LANES)

    def group_body(g, c):
        acc = jnp.zeros((LANES,), jnp.float32)
        for rr in range(LANES):
            r = g * LANES + rr
            u0 = rows[r, pl.ds(0, LANES)]
            u1 = rows[r, pl.ds(LANES, LANES)]
            v0 = rows[r, pl.ds(EMBED_K, LANES)]
            v1 = rows[r, pl.ds(EMBED_K + LANES, LANES)]
            s = jnp.sum(u0 * v0 + u1 * v1)
            acc = jnp.where(lane == rr, s, acc)
        out_v[pl.ds(g * LANES, LANES)] = acc
        return c

    lax.fori_loop(0, GROUPS, group_body, 0)

    pltpu.sync_copy(out_v, out_hbm.at[pl.ds(base, ROWS_PER_W)])


@jax.jit
def kernel(x, W, H):
    uidx = x[:, 0].astype(jnp.int32)
    iidx = x[:, 1].astype(jnp.int32)

    mesh = plsc.VectorSubcoreMesh(core_axis_name="c", subcore_axis_name="s")
    run = functools.partial(
        pl.kernel,
        out_type=jax.ShapeDtypeStruct((BATCH,), jnp.float32),
        mesh=mesh,
        compiler_params=pltpu.CompilerParams(
            needs_layout_passes=False, use_tc_tiling_on_sc=True),
        scratch_types=[
            pltpu.VMEM((ROWS_PER_W,), jnp.int32),
            pltpu.VMEM((ROWS_PER_W,), jnp.int32),
            pltpu.VMEM((ROWS_PER_W, 4 * EMBED_K), jnp.float32),
            pltpu.VMEM((ROWS_PER_W,), jnp.float32),
            pltpu.SemaphoreType.DMA,
        ],
    )(_sc_body)
    return run(uidx, iidx, W, H)


# speedup vs baseline: 1.5041x; 1.0019x over previous
import functools

import jax
import jax.numpy as jnp
from jax import lax
from jax.experimental import pallas as pl
from jax.experimental.pallas import tpu as pltpu
from jax.experimental.pallas import tpu_sc as plsc

BATCH = 16384
EMBED_K = 32
LANES = 16
NUM_WORKERS = 32
ROWS_PER_W = BATCH // NUM_WORKERS
GROUPS = ROWS_PER_W // LANES
WINDOW_STEPS = 2


def _sc_body(uidx_hbm, iidx_hbm, w_hbm, h_hbm, out_hbm,
             idx_u, idx_i, rows, out_v, sem):
    nc = 2
    wid = lax.axis_index("s") * nc + lax.axis_index("c")
    base = wid * ROWS_PER_W

    pltpu.sync_copy(uidx_hbm.at[pl.ds(base, ROWS_PER_W)], idx_u)
    pltpu.sync_copy(iidx_hbm.at[pl.ds(base, ROWS_PER_W)], idx_i)

    def fire_step(t):
        vu = idx_u[pl.ds(t * LANES, LANES)]
        vi = idx_i[pl.ds(t * LANES, LANES)]
        for k in range(LANES):
            j = t * LANES + k
            pltpu.async_copy(
                w_hbm.at[vu[k]], rows.at[j, pl.ds(0, EMBED_K)], sem)
            pltpu.async_copy(
                h_hbm.at[vi[k]], rows.at[j, pl.ds(EMBED_K, EMBED_K)], sem)

    def drain_step():
        for _ in range(2 * LANES):
            pltpu.make_async_copy(
                w_hbm.at[0], rows.at[0, pl.ds(0, EMBED_K)], sem).wait()

    nsteps = ROWS_PER_W // LANES

    @plsc.parallel_loop(0, nsteps, unroll=2)
    def _fire_all(t):
        fire_step(t)

    def final_drain(t, c):
        drain_step()
        return c
    lax.fori_loop(0, nsteps, final_drain, 0)

    lane = lax.iota(jnp.int32, LANES)

    def group_body(g, c):
        acc = jnp.zeros((LANES,), jnp.float32)
        for rr in range(LANES):
            r = g * LANES + rr
            u0 = rows[r, pl.ds(0, LANES)]
            u1 = rows[r, pl.ds(LANES, LANES)]
            v0 = rows[r, pl.ds(EMBED_K, LANES)]
            v1 = rows[r, pl.ds(EMBED_K + LANES, LANES)]
            s = jnp.sum(u0 * v0 + u1 * v1)
            acc = jnp.where(lane == rr, s, acc)
        out_v[pl.ds(g * LANES, LANES)] = acc
        return c

    lax.fori_loop(0, GROUPS, group_body, 0)

    pltpu.sync_copy(out_v, out_hbm.at[pl.ds(base, ROWS_PER_W)])


@jax.jit
def kernel(x, W, H):
    uidx = x[:, 0].astype(jnp.int32)
    iidx = x[:, 1].astype(jnp.int32)

    mesh = plsc.VectorSubcoreMesh(core_axis_name="c", subcore_axis_name="s")
    run = functools.partial(
        pl.kernel,
        out_type=jax.ShapeDtypeStruct((BATCH,), jnp.float32),
        mesh=mesh,
        compiler_params=pltpu.CompilerParams(
            needs_layout_passes=False, use_tc_tiling_on_sc=True),
        scratch_types=[
            pltpu.VMEM((ROWS_PER_W,), jnp.int32),
            pltpu.VMEM((ROWS_PER_W,), jnp.int32),
            pltpu.VMEM((ROWS_PER_W, 4 * EMBED_K), jnp.float32),
            pltpu.VMEM((ROWS_PER_W,), jnp.float32),
            pltpu.SemaphoreType.DMA,
        ],
    )(_sc_body)
    return run(uidx, iidx, W, H)
